# Initial kernel scaffold; baseline (speedup 1.0000x reference)
import jax
import jax.numpy as jnp
from jax.experimental import pallas as pl


def kernel(x, edge_index, w1, w2):
    raise NotImplementedError("write your pallas kernel here")


# speedup vs baseline: 19.4530x; 19.4530x over previous
import functools

import jax
import jax.numpy as jnp
from jax import lax
from jax.experimental import pallas as pl
from jax.experimental.pallas import tpu as pltpu
from jax.experimental.pallas import tpu_sc as plsc

N = 10000
D = 128
DH = 64
E = 320000
B = 100
ROWS = E // B
NC, NS = 2, 16
NPAD = 10240
STRIPE = NPAD // NS
TAIL = N - (NS - 1) * STRIPE
G = 8
GP = 4

_mesh = plsc.VectorSubcoreMesh(
    core_axis_name="c", subcore_axis_name="s", num_cores=NC, num_subcores=NS)
_sc_params = pltpu.CompilerParams(use_tc_tiling_on_sc=False)


@functools.partial(
    pl.kernel,
    out_type=jax.ShapeDtypeStruct((NPAD,), jnp.float32),
    mesh=_mesh,
    compiler_params=_sc_params,
    scratch_types=[
        pltpu.VMEM((STRIPE,), jnp.float32),
        pltpu.VMEM((112,), jnp.float32),
        pltpu.VMEM((G, B), jnp.int32),
        pltpu.VMEM_SHARED((NPAD,), jnp.float32),
    ],
)
def _sc_deg(dst_hbm, deg_out, zbuf, ones_v, didx, deg_sh):
    c = lax.axis_index("c")
    s = lax.axis_index("s")
    for i in range(STRIPE // 16):
        zbuf[pl.ds(16 * i, 16)] = jnp.zeros((16,), jnp.float32)
    for i in range(112 // 16):
        ones_v[pl.ds(16 * i, 16)] = jnp.ones((16,), jnp.float32)
    pltpu.sync_copy(zbuf, deg_sh.at[pl.ds(STRIPE * s, STRIPE)])
    plsc.subcore_barrier()

    rows_per_t = ROWS // NS
    base = s * rows_per_t

    @pl.when(c == 0)
    def _():
        def body(g, carry):
            rb = base + g * G
            pltpu.sync_copy(dst_hbm.at[pl.ds(rb, G)], didx)
            for j in range(G):
                pltpu.sync_copy(ones_v.at[pl.ds(0, B)],
                                deg_sh.at[didx.at[j]], add=True)
            return carry

        lax.fori_loop(0, rows_per_t // G, body, 0)

    plsc.subcore_barrier()

    @pl.when(c == 0)
    def _():
        pltpu.sync_copy(deg_sh.at[pl.ds(STRIPE * s, STRIPE)],
                        deg_out.at[pl.ds(STRIPE * s, STRIPE)])


@functools.partial(
    pl.kernel,
    out_type=jax.ShapeDtypeStruct((N, D), jnp.float32),
    mesh=_mesh,
    compiler_params=_sc_params,
    scratch_types=[
        pltpu.VMEM((GP, B), jnp.int32),
        pltpu.VMEM((GP, B), jnp.int32),
        pltpu.VMEM((GP, B, DH), jnp.float32),
        pltpu.VMEM_SHARED((NPAD, DH), jnp.float32),
        pltpu.VMEM_SHARED((NPAD, DH), jnp.float32),
        pltpu.SemaphoreType.DMA,
        pltpu.SemaphoreType.DMA,
    ],
)
def _sc_prop(u_hbm, src_hbm, dst_hbm, p_out, sidx, didx, rows, h_sh, acc_sh,
             gsem, ssem):
    c = lax.axis_index("c")
    s = lax.axis_index("s")
    colb = c * DH

    @pl.when(s < NS - 1)
    def _():
        rb = STRIPE * s
        pltpu.sync_copy(u_hbm.at[pl.ds(rb, STRIPE), pl.ds(colb, DH)],
                        h_sh.at[pl.ds(rb, STRIPE)])
        pltpu.sync_copy(u_hbm.at[pl.ds(rb, STRIPE), pl.ds(colb, DH)],
                        acc_sh.at[pl.ds(rb, STRIPE)])

    @pl.when(s == NS - 1)
    def _():
        rb = STRIPE * (NS - 1)
        pltpu.sync_copy(u_hbm.at[pl.ds(rb, TAIL), pl.ds(colb, DH)],
                        h_sh.at[pl.ds(rb, TAIL)])
        pltpu.sync_copy(u_hbm.at[pl.ds(rb, TAIL), pl.ds(colb, DH)],
                        acc_sh.at[pl.ds(rb, TAIL)])

    plsc.subcore_barrier()

    rows_per_t = ROWS // NS
    base = s * rows_per_t

    def body(g, carry):
        rb = base + g * GP
        pltpu.sync_copy(src_hbm.at[pl.ds(rb, GP)], sidx)
        pltpu.sync_copy(dst_hbm.at[pl.ds(rb, GP)], didx)
        gd = [pltpu.async_copy(h_sh.at[sidx.at[j]], rows.at[j], gsem)
              for j in range(GP)]
        for j in range(GP):
            gd[j].wait()
        sd = [pltpu.async_copy(rows.at[j], acc_sh.at[didx.at[j]], ssem,
                               add=True)
              for j in range(GP)]
        for j in range(GP):
            sd[j].wait()
        return carry

    lax.fori_loop(0, rows_per_t // GP, body, 0)
    plsc.subcore_barrier()

    @pl.when(s < NS - 1)
    def _():
        rb = STRIPE * s
        pltpu.sync_copy(acc_sh.at[pl.ds(rb, STRIPE)],
                        p_out.at[pl.ds(rb, STRIPE), pl.ds(colb, DH)])

    @pl.when(s == NS - 1)
    def _():
        rb = STRIPE * (NS - 1)
        pltpu.sync_copy(acc_sh.at[pl.ds(rb, TAIL)],
                        p_out.at[pl.ds(rb, TAIL), pl.ds(colb, DH)])


def _tc1_body(x_ref, w_ref, deg_ref, u_ref, dinv_ref):
    dinv = lax.rsqrt(deg_ref[...] + 1.0)
    m = lax.dot_general(x_ref[...], w_ref[...], (((1,), (1,)), ((), ())),
                        preferred_element_type=jnp.float32)
    u_ref[...] = m * dinv
    dinv_ref[...] = dinv


def _tc_mid_body(p_ref, w_ref, dinv_ref, u_ref):
    dinv = dinv_ref[...]
    m = lax.dot_general(p_ref[...], w_ref[...], (((1,), (1,)), ((), ())),
                        preferred_element_type=jnp.float32)
    u_ref[...] = m * (dinv * dinv)


def _tc_out_body(p_ref, dinv_ref, o_ref):
    o_ref[...] = p_ref[...] * dinv_ref[...]


_tc1 = pl.pallas_call(
    _tc1_body,
    out_shape=[jax.ShapeDtypeStruct((N, D), jnp.float32),
               jax.ShapeDtypeStruct((N, 1), jnp.float32)])

_tc_mid = pl.pallas_call(
    _tc_mid_body,
    out_shape=jax.ShapeDtypeStruct((N, D), jnp.float32))

_tc_out = pl.pallas_call(
    _tc_out_body,
    out_shape=jax.ShapeDtypeStruct((N, D), jnp.float32))


def kernel(x, edge_index, w1, w2):
    src2d = edge_index[0].reshape(ROWS, B)
    dst2d = edge_index[1].reshape(ROWS, B)
    deg = _sc_deg(dst2d)[:N].reshape(N, 1)
    u1, dinv = _tc1(x, w1, deg)
    p1 = _sc_prop(u1, src2d, dst2d)
    u2 = _tc_mid(p1, w1, dinv)
    p2 = _sc_prop(u2, src2d, dst2d)
    u3 = _tc_mid(p2, w2, dinv)
    p3 = _sc_prop(u3, src2d, dst2d)
    return _tc_out(p3, dinv)


# speedup vs baseline: 22.7750x; 1.1708x over previous
import functools

import jax
import jax.numpy as jnp
from jax import lax
from jax.experimental import pallas as pl
from jax.experimental.pallas import tpu as pltpu
from jax.experimental.pallas import tpu_sc as plsc

N = 10000
D = 128
DH = 64
E = 320000
B = 100
ROWS = E // B
NC, NS = 2, 16
NPAD = 10240
STRIPE = NPAD // NS
TAIL = N - (NS - 1) * STRIPE
G = 8
GP = 5
PAIRS = ROWS // NS // (2 * GP)

_mesh = plsc.VectorSubcoreMesh(
    core_axis_name="c", subcore_axis_name="s", num_cores=NC, num_subcores=NS)
_sc_params = pltpu.CompilerParams(use_tc_tiling_on_sc=False)


@functools.partial(
    pl.kernel,
    out_type=jax.ShapeDtypeStruct((NPAD,), jnp.float32),
    mesh=_mesh,
    compiler_params=_sc_params,
    scratch_types=[
        pltpu.VMEM((STRIPE,), jnp.float32),
        pltpu.VMEM((112,), jnp.float32),
        pltpu.VMEM((G, B), jnp.int32),
        pltpu.VMEM_SHARED((NPAD,), jnp.float32),
    ],
)
def _sc_deg(dst_hbm, deg_out, zbuf, ones_v, didx, deg_sh):
    c = lax.axis_index("c")
    s = lax.axis_index("s")
    for i in range(STRIPE // 16):
        zbuf[pl.ds(16 * i, 16)] = jnp.zeros((16,), jnp.float32)
    for i in range(112 // 16):
        ones_v[pl.ds(16 * i, 16)] = jnp.ones((16,), jnp.float32)
    pltpu.sync_copy(zbuf, deg_sh.at[pl.ds(STRIPE * s, STRIPE)])
    plsc.subcore_barrier()

    rows_per_t = ROWS // NS
    base = s * rows_per_t

    @pl.when(c == 0)
    def _():
        def body(g, carry):
            rb = base + g * G
            pltpu.sync_copy(dst_hbm.at[pl.ds(rb, G)], didx)
            for j in range(G):
                pltpu.sync_copy(ones_v.at[pl.ds(0, B)],
                                deg_sh.at[didx.at[j]], add=True)
            return carry

        lax.fori_loop(0, rows_per_t // G, body, 0)

    plsc.subcore_barrier()

    @pl.when(c == 0)
    def _():
        pltpu.sync_copy(deg_sh.at[pl.ds(STRIPE * s, STRIPE)],
                        deg_out.at[pl.ds(STRIPE * s, STRIPE)])


@functools.partial(
    pl.kernel,
    out_type=jax.ShapeDtypeStruct((N, D), jnp.float32),
    mesh=_mesh,
    compiler_params=_sc_params,
    scratch_types=[
        pltpu.VMEM((2, GP, B), jnp.int32),
        pltpu.VMEM((2, GP, B), jnp.int32),
        pltpu.VMEM((2, GP, B, DH), jnp.float32),
        pltpu.VMEM_SHARED((NPAD, DH), jnp.float32),
        pltpu.SemaphoreType.DMA,
        pltpu.SemaphoreType.DMA,
        pltpu.SemaphoreType.DMA,
        pltpu.SemaphoreType.DMA,
    ],
)
def _sc_prop(ucat_hbm, src_hbm, dst_hbm, p_out, sidx, didx, rows, acc_sh,
             gsemA, gsemB, ssemA, ssemB):
    c = lax.axis_index("c")
    s = lax.axis_index("s")
    colb = c * DH
    gsem = [gsemA, gsemB]
    ssem = [ssemA, ssemB]

    @pl.when(s < NS - 1)
    def _():
        pltpu.sync_copy(ucat_hbm.at[pl.ds(c * N + STRIPE * s, STRIPE)],
                        acc_sh.at[pl.ds(STRIPE * s, STRIPE)])

    @pl.when(s == NS - 1)
    def _():
        pltpu.sync_copy(ucat_hbm.at[pl.ds(c * N + STRIPE * s, TAIL)],
                        acc_sh.at[pl.ds(STRIPE * s, TAIL)])

    plsc.subcore_barrier()

    rows_per_t = ROWS // NS
    base = s * rows_per_t

    def load_and_gather(ab, rb):
        pltpu.sync_copy(src_hbm.at[c, pl.ds(rb, GP)], sidx.at[ab])
        pltpu.sync_copy(dst_hbm.at[pl.ds(rb, GP)], didx.at[ab])
        return [pltpu.async_copy(ucat_hbm.at[sidx.at[ab, j]], rows.at[ab, j],
                                 gsem[ab])
                for j in range(GP)]

    def scatter(ab):
        return [pltpu.async_copy(rows.at[ab, j], acc_sh.at[didx.at[ab, j]],
                                 ssem[ab], add=True)
                for j in range(GP)]

    def drain(descs):
        for d in descs:
            d.wait()

    def drain_gathers(ab):
        drain([pltpu.make_async_copy(ucat_hbm.at[sidx.at[ab, j]],
                                     rows.at[ab, j], gsem[ab])
               for j in range(GP)])

    load_and_gather(0, base)

    def body(k, carry):
        drain_gathers(0)
        sa = scatter(0)
        gb = load_and_gather(1, base + (2 * k + 1) * GP)
        drain(gb)
        sb = scatter(1)
        drain(sa)

        @pl.when(k < PAIRS - 1)
        def _():
            load_and_gather(0, base + (2 * k + 2) * GP)

        drain(sb)
        return carry

    lax.fori_loop(0, PAIRS, body, 0)
    plsc.subcore_barrier()

    @pl.when(s < NS - 1)
    def _():
        rb = STRIPE * s
        pltpu.sync_copy(acc_sh.at[pl.ds(rb, STRIPE)],
                        p_out.at[pl.ds(rb, STRIPE), pl.ds(colb, DH)])

    @pl.when(s == NS - 1)
    def _():
        rb = STRIPE * (NS - 1)
        pltpu.sync_copy(acc_sh.at[pl.ds(rb, TAIL)],
                        p_out.at[pl.ds(rb, TAIL), pl.ds(colb, DH)])


def _tc1_body(x_ref, w_ref, deg_ref, ucat_ref, dinv_ref):
    dinv = lax.rsqrt(deg_ref[...] + 1.0)
    m = lax.dot_general(x_ref[...], w_ref[...], (((1,), (1,)), ((), ())),
                        preferred_element_type=jnp.float32)
    u = m * dinv
    ucat_ref[pl.ds(0, N), :] = u[:, :DH]
    ucat_ref[pl.ds(N, N), :] = u[:, DH:]
    dinv_ref[...] = dinv


def _tc_mid_body(p_ref, w_ref, dinv_ref, ucat_ref):
    dinv = dinv_ref[...]
    m = lax.dot_general(p_ref[...], w_ref[...], (((1,), (1,)), ((), ())),
                        preferred_element_type=jnp.float32)
    u = m * (dinv * dinv)
    ucat_ref[pl.ds(0, N), :] = u[:, :DH]
    ucat_ref[pl.ds(N, N), :] = u[:, DH:]


def _tc_out_body(p_ref, dinv_ref, o_ref):
    o_ref[...] = p_ref[...] * dinv_ref[...]


_tc1 = pl.pallas_call(
    _tc1_body,
    out_shape=[jax.ShapeDtypeStruct((2 * N, DH), jnp.float32),
               jax.ShapeDtypeStruct((N, 1), jnp.float32)])

_tc_mid = pl.pallas_call(
    _tc_mid_body,
    out_shape=jax.ShapeDtypeStruct((2 * N, DH), jnp.float32))

_tc_out = pl.pallas_call(
    _tc_out_body,
    out_shape=jax.ShapeDtypeStruct((N, D), jnp.float32))


def kernel(x, edge_index, w1, w2):
    src = edge_index[0]
    srcstk = jnp.stack([src, src + N]).reshape(NC, ROWS, B)
    dst2d = edge_index[1].reshape(ROWS, B)
    deg = _sc_deg(dst2d)[:N].reshape(N, 1)
    u1, dinv = _tc1(x, w1, deg)
    p1 = _sc_prop(u1, srcstk, dst2d)
    u2 = _tc_mid(p1, w1, dinv)
    p2 = _sc_prop(u2, srcstk, dst2d)
    u3 = _tc_mid(p2, w2, dinv)
    p3 = _sc_prop(u3, srcstk, dst2d)
    return _tc_out(p3, dinv)


# speedup vs baseline: 23.8875x; 1.0488x over previous
import functools

import jax
import jax.numpy as jnp
from jax import lax
from jax.experimental import pallas as pl
from jax.experimental.pallas import tpu as pltpu
from jax.experimental.pallas import tpu_sc as plsc

N = 10000
D = 128
DH = 64
E = 320000
B = 128
ROWS = 2560
EPAD = ROWS * B - E
NC, NS = 2, 16
NPAD = 10240
STRIPE = NPAD // NS
TAIL = N - (NS - 1) * STRIPE
G = 8
GP = 4
RPT = ROWS // NS
PAIRS = RPT // (2 * GP)

_mesh = plsc.VectorSubcoreMesh(
    core_axis_name="c", subcore_axis_name="s", num_cores=NC, num_subcores=NS)
_sc_params = pltpu.CompilerParams(use_tc_tiling_on_sc=False)


@functools.partial(
    pl.kernel,
    out_type=jax.ShapeDtypeStruct((NC, NPAD), jnp.float32),
    mesh=_mesh,
    compiler_params=_sc_params,
    scratch_types=[
        pltpu.VMEM((STRIPE,), jnp.float32),
        pltpu.VMEM((B,), jnp.float32),
        pltpu.VMEM((G, B), jnp.int32),
        pltpu.VMEM_SHARED((NPAD,), jnp.float32),
        pltpu.SemaphoreType.DMA,
    ],
)
def _sc_deg(dst_hbm, deg_out, zbuf, ones_v, didx, deg_sh, sem):
    c = lax.axis_index("c")
    s = lax.axis_index("s")
    for i in range(STRIPE // 16):
        zbuf[pl.ds(16 * i, 16)] = jnp.zeros((16,), jnp.float32)
    for i in range(B // 16):
        ones_v[pl.ds(16 * i, 16)] = jnp.ones((16,), jnp.float32)
    pltpu.sync_copy(zbuf, deg_sh.at[pl.ds(STRIPE * s, STRIPE)])
    plsc.subcore_barrier()

    rows_per_w = ROWS // (NC * NS)
    base = (c * NS + s) * rows_per_w

    def body(g, carry):
        rb = base + g * G
        pltpu.sync_copy(dst_hbm.at[pl.ds(rb, G)], didx)
        sd = [pltpu.async_copy(ones_v, deg_sh.at[didx.at[j]], sem, add=True)
              for j in range(G)]
        for d in sd:
            d.wait()
        return carry

    lax.fori_loop(0, rows_per_w // G, body, 0)
    plsc.subcore_barrier()
    pltpu.sync_copy(deg_sh.at[pl.ds(STRIPE * s, STRIPE)],
                    deg_out.at[c, pl.ds(STRIPE * s, STRIPE)])


@functools.partial(
    pl.kernel,
    out_type=jax.ShapeDtypeStruct((N, D), jnp.float32),
    mesh=_mesh,
    compiler_params=_sc_params,
    scratch_types=[
        pltpu.VMEM((2, GP, B), jnp.int32),
        pltpu.VMEM((2, GP, B), jnp.int32),
        pltpu.VMEM((2, GP, B, DH), jnp.float32),
        pltpu.VMEM_SHARED((NPAD, DH), jnp.float32),
        pltpu.SemaphoreType.DMA,
        pltpu.SemaphoreType.DMA,
        pltpu.SemaphoreType.DMA,
        pltpu.SemaphoreType.DMA,
    ],
)
def _sc_prop(ucat_hbm, src_hbm, dst_hbm, p_out, sidx, didx, rows, acc_sh,
             gsemA, gsemB, ssemA, ssemB):
    c = lax.axis_index("c")
    s = lax.axis_index("s")
    colb = c * DH
    gsem = [gsemA, gsemB]
    ssem = [ssemA, ssemB]

    @pl.when(s < NS - 1)
    def _():
        pltpu.sync_copy(ucat_hbm.at[pl.ds(c * NPAD + STRIPE * s, STRIPE)],
                        acc_sh.at[pl.ds(STRIPE * s, STRIPE)])

    @pl.when(s == NS - 1)
    def _():
        pltpu.sync_copy(ucat_hbm.at[pl.ds(c * NPAD + STRIPE * s, TAIL)],
                        acc_sh.at[pl.ds(STRIPE * s, TAIL)])

    plsc.subcore_barrier()

    base = s * RPT

    def load_and_gather(ab, rb):
        pltpu.sync_copy(src_hbm.at[c, pl.ds(rb, GP)], sidx.at[ab])
        pltpu.sync_copy(dst_hbm.at[pl.ds(rb, GP)], didx.at[ab])
        return [pltpu.async_copy(ucat_hbm.at[sidx.at[ab, j]], rows.at[ab, j],
                                 gsem[ab])
                for j in range(GP)]

    def scatter(ab):
        return [pltpu.async_copy(rows.at[ab, j], acc_sh.at[didx.at[ab, j]],
                                 ssem[ab], add=True)
                for j in range(GP)]

    def drain(descs):
        for d in descs:
            d.wait()

    def drain_gathers(ab):
        drain([pltpu.make_async_copy(ucat_hbm.at[sidx.at[ab, j]],
                                     rows.at[ab, j], gsem[ab])
               for j in range(GP)])

    load_and_gather(0, base)

    def body(k, carry):
        drain_gathers(0)
        sa = scatter(0)
        gb = load_and_gather(1, base + (2 * k + 1) * GP)
        drain(gb)
        sb = scatter(1)
        drain(sa)

        @pl.when(k < PAIRS - 1)
        def _():
            load_and_gather(0, base + (2 * k + 2) * GP)

        drain(sb)
        return carry

    lax.fori_loop(0, PAIRS, body, 0)
    plsc.subcore_barrier()

    @pl.when(s < NS - 1)
    def _():
        rb = STRIPE * s
        pltpu.sync_copy(acc_sh.at[pl.ds(rb, STRIPE)],
                        p_out.at[pl.ds(rb, STRIPE), pl.ds(colb, DH)])

    @pl.when(s == NS - 1)
    def _():
        rb = STRIPE * (NS - 1)
        pltpu.sync_copy(acc_sh.at[pl.ds(rb, TAIL)],
                        p_out.at[pl.ds(rb, TAIL), pl.ds(colb, DH)])


def _split_u(u, ucat_ref):
    ucat_ref[pl.ds(0, N), :] = u[:, :DH]
    ucat_ref[pl.ds(NPAD, N), :] = u[:, DH:]


def _tc1_body(x_ref, w_ref, deg_ref, ucat_ref, dinv_ref):
    deg2 = deg_ref[...][:, :N]
    ones = jnp.ones((NC, 1), jnp.float32)
    degsum = lax.dot_general(deg2, ones, (((0,), (0,)), ((), ())),
                             preferred_element_type=jnp.float32)
    dinv = lax.rsqrt(degsum + 1.0)
    m = lax.dot_general(x_ref[...], w_ref[...], (((1,), (1,)), ((), ())),
                        preferred_element_type=jnp.float32)
    _split_u(m * dinv, ucat_ref)
    dinv_ref[...] = dinv


def _tc_mid_body(p_ref, w_ref, dinv_ref, ucat_ref):
    dinv = dinv_ref[...]
    m = lax.dot_general(p_ref[...], w_ref[...], (((1,), (1,)), ((), ())),
                        preferred_element_type=jnp.float32)
    _split_u(m * (dinv * dinv), ucat_ref)


def _tc_out_body(p_ref, dinv_ref, o_ref):
    o_ref[...] = p_ref[...] * dinv_ref[...]


_tc1 = pl.pallas_call(
    _tc1_body,
    out_shape=[jax.ShapeDtypeStruct((2 * NPAD, DH), jnp.float32),
               jax.ShapeDtypeStruct((N, 1), jnp.float32)])

_tc_mid = pl.pallas_call(
    _tc_mid_body,
    out_shape=jax.ShapeDtypeStruct((2 * NPAD, DH), jnp.float32))

_tc_out = pl.pallas_call(
    _tc_out_body,
    out_shape=jax.ShapeDtypeStruct((N, D), jnp.float32))


def kernel(x, edge_index, w1, w2):
    pad = (jnp.arange(EPAD, dtype=jnp.int32) % (NPAD - N)) + N
    src = jnp.concatenate([edge_index[0], pad])
    dst = jnp.concatenate([edge_index[1], pad])
    srcstk = jnp.stack([src, src + NPAD]).reshape(NC, ROWS, B)
    dst2d = dst.reshape(ROWS, B)
    deg = _sc_deg(dst2d)
    u1, dinv = _tc1(x, w1, deg)
    p1 = _sc_prop(u1, srcstk, dst2d)
    u2 = _tc_mid(p1, w1, dinv)
    p2 = _sc_prop(u2, srcstk, dst2d)
    u3 = _tc_mid(p2, w2, dinv)
    p3 = _sc_prop(u3, srcstk, dst2d)
    return _tc_out(p3, dinv)


# speedup vs baseline: 28.0136x; 1.1727x over previous
import functools

import jax
import jax.numpy as jnp
from jax import lax
from jax.experimental import pallas as pl
from jax.experimental.pallas import tpu as pltpu
from jax.experimental.pallas import tpu_sc as plsc

N = 10000
D = 128
DH = 64
E = 320000
B = 128
ROWS = 2560
EPAD = ROWS * B - E
NC, NS = 2, 16
NPAD = 10240
STRIPE = NPAD // NS
TAIL = N - (NS - 1) * STRIPE
G = 8
GP = 5
RPT = ROWS // NS
PAIRS = RPT // (2 * GP)
RB = 1000
L = 16

_mesh = plsc.VectorSubcoreMesh(
    core_axis_name="c", subcore_axis_name="s", num_cores=NC, num_subcores=NS)
_sc_params = pltpu.CompilerParams(use_tc_tiling_on_sc=False)


@functools.partial(
    pl.kernel,
    out_type=jax.ShapeDtypeStruct((NC, NPAD), jnp.float32),
    mesh=_mesh,
    compiler_params=_sc_params,
    scratch_types=[
        pltpu.VMEM((STRIPE,), jnp.float32),
        pltpu.VMEM((B,), jnp.float32),
        pltpu.VMEM((G, B), jnp.int32),
        pltpu.VMEM_SHARED((NPAD,), jnp.float32),
        pltpu.SemaphoreType.DMA,
    ],
)
def _sc_deg(dst_hbm, deg_out, zbuf, ones_v, didx, deg_sh, sem):
    c = lax.axis_index("c")
    s = lax.axis_index("s")
    for i in range(STRIPE // L):
        zbuf[pl.ds(L * i, L)] = jnp.zeros((L,), jnp.float32)
    for i in range(B // L):
        ones_v[pl.ds(L * i, L)] = jnp.ones((L,), jnp.float32)
    pltpu.sync_copy(zbuf, deg_sh.at[pl.ds(STRIPE * s, STRIPE)])
    plsc.subcore_barrier()

    rows_per_w = ROWS // (NC * NS)
    base = (c * NS + s) * rows_per_w

    def body(g, carry):
        rb = base + g * G
        pltpu.sync_copy(dst_hbm.at[pl.ds(rb, G)], didx)
        sd = [pltpu.async_copy(ones_v, deg_sh.at[didx.at[j]], sem, add=True)
              for j in range(G)]
        for d in sd:
            d.wait()
        return carry

    lax.fori_loop(0, rows_per_w // G, body, 0)
    plsc.subcore_barrier()
    pltpu.sync_copy(deg_sh.at[pl.ds(STRIPE * s, STRIPE)],
                    deg_out.at[c, pl.ds(STRIPE * s, STRIPE)])


@functools.partial(
    pl.kernel,
    out_type=jax.ShapeDtypeStruct((N, D), jnp.float32),
    mesh=_mesh,
    compiler_params=_sc_params,
    scratch_types=[
        pltpu.VMEM((2, GP, 2, B), jnp.int32),
        pltpu.VMEM((2, GP, B, DH), jnp.float32),
        pltpu.VMEM_SHARED((NPAD, DH), jnp.float32),
        pltpu.SemaphoreType.DMA,
        pltpu.SemaphoreType.DMA,
        pltpu.SemaphoreType.DMA,
        pltpu.SemaphoreType.DMA,
    ],
)
def _sc_prop(ucat_hbm, edge_hbm, zeros_hbm, p_out, exbuf, rows, acc_sh,
             gsemA, gsemB, ssemA, ssemB):
    c = lax.axis_index("c")
    s = lax.axis_index("s")
    colb = c * DH
    gsem = [gsemA, gsemB]
    ssem = [ssemA, ssemB]

    pltpu.sync_copy(zeros_hbm, acc_sh.at[pl.ds(STRIPE * s, STRIPE)])
    plsc.subcore_barrier()

    base = s * RPT

    def load_and_gather(ab, rb):
        pltpu.sync_copy(edge_hbm.at[c, pl.ds(rb, GP)], exbuf.at[ab])
        return [pltpu.async_copy(ucat_hbm.at[exbuf.at[ab, j, 0]],
                                 rows.at[ab, j], gsem[ab])
                for j in range(GP)]

    def scatter(ab):
        return [pltpu.async_copy(rows.at[ab, j],
                                 acc_sh.at[exbuf.at[ab, j, 1]],
                                 ssem[ab], add=True)
                for j in range(GP)]

    def drain(descs):
        for d in descs:
            d.wait()

    def drain_gathers(ab):
        drain([pltpu.make_async_copy(ucat_hbm.at[exbuf.at[ab, j, 0]],
                                     rows.at[ab, j], gsem[ab])
               for j in range(GP)])

    load_and_gather(0, base)

    def body(k, carry):
        drain_gathers(0)
        sa = scatter(0)
        gb = load_and_gather(1, base + (2 * k + 1) * GP)
        drain(gb)
        sb = scatter(1)
        drain(sa)

        @pl.when(k < PAIRS - 1)
        def _():
            load_and_gather(0, base + (2 * k + 2) * GP)

        drain(sb)
        return carry

    lax.fori_loop(0, PAIRS, body, 0)
    plsc.subcore_barrier()

    @pl.when(s < NS - 1)
    def _():
        rb = STRIPE * s
        pltpu.sync_copy(acc_sh.at[pl.ds(rb, STRIPE)],
                        p_out.at[pl.ds(rb, STRIPE), pl.ds(colb, DH)])

    @pl.when(s == NS - 1)
    def _():
        rb = STRIPE * (NS - 1)
        pltpu.sync_copy(acc_sh.at[pl.ds(rb, TAIL)],
                        p_out.at[pl.ds(rb, TAIL), pl.ds(colb, DH)])


def _tc1_body(x_ref, w_ref, deg_ref, v_ref, dinv_ref):
    degsum = jnp.sum(deg_ref[...], axis=1, keepdims=True)
    dinv = lax.rsqrt(degsum + 1.0)
    m = lax.dot_general(x_ref[...], w_ref[...], (((1,), (1,)), ((), ())),
                        preferred_element_type=jnp.float32)
    v_ref[...] = m * dinv
    dinv_ref[...] = dinv


def _tc_mid_body(s_ref, v_ref, w_ref, dinv_ref, vo_ref):
    dinv = dinv_ref[...]
    p = s_ref[...] + v_ref[...]
    m = lax.dot_general(p, w_ref[...], (((1,), (1,)), ((), ())),
                        preferred_element_type=jnp.float32)
    vo_ref[...] = m * (dinv * dinv)


def _tc_out_body(s_ref, v_ref, dinv_ref, o_ref):
    o_ref[...] = (s_ref[...] + v_ref[...]) * dinv_ref[...]


_row_spec = pl.BlockSpec((RB, D), lambda i: (i, 0))
_w_spec = pl.BlockSpec((D, D), lambda i: (0, 0))
_dinv_spec = pl.BlockSpec((RB, 1), lambda i: (i, 0))

_tc1 = pl.pallas_call(
    _tc1_body,
    grid=(N // RB,),
    in_specs=[_row_spec, _w_spec, pl.BlockSpec((RB, NC), lambda i: (i, 0))],
    out_specs=[_row_spec, _dinv_spec],
    out_shape=[jax.ShapeDtypeStruct((N, D), jnp.float32),
               jax.ShapeDtypeStruct((N, 1), jnp.float32)])

_tc_mid = pl.pallas_call(
    _tc_mid_body,
    grid=(N // RB,),
    in_specs=[_row_spec, _row_spec, _w_spec, _dinv_spec],
    out_specs=_row_spec,
    out_shape=jax.ShapeDtypeStruct((N, D), jnp.float32))

_tc_out = pl.pallas_call(
    _tc_out_body,
    grid=(N // RB,),
    in_specs=[_row_spec, _row_spec, _dinv_spec],
    out_specs=_row_spec,
    out_shape=jax.ShapeDtypeStruct((N, D), jnp.float32))


def kernel(x, edge_index, w1, w2):
    pad = jnp.arange(EPAD, dtype=jnp.int32) % (NPAD - N)
    src2d = jnp.concatenate([edge_index[0], pad]).reshape(ROWS, B)
    dst2d = jnp.concatenate([edge_index[1], pad + N]).reshape(ROWS, B)
    edges = jnp.stack([jnp.stack([2 * src2d, dst2d], axis=1),
                       jnp.stack([2 * src2d + 1, dst2d], axis=1)])
    zeros = jnp.zeros((STRIPE, DH), jnp.float32)
    deg = _sc_deg(dst2d).T
    v1, dinv = _tc1(x, w1, deg)
    s1 = _sc_prop(v1.reshape(2 * N, DH), edges, zeros)
    v2 = _tc_mid(s1, v1, w1, dinv)
    s2 = _sc_prop(v2.reshape(2 * N, DH), edges, zeros)
    v3 = _tc_mid(s2, v2, w2, dinv)
    s3 = _sc_prop(v3.reshape(2 * N, DH), edges, zeros)
    return _tc_out(s3, v3, dinv)
